# speedup vs baseline: 2.7577x; 2.7577x over previous
import functools

import jax
import jax.numpy as jnp
from jax import lax
from jax.experimental import pallas as pl
from jax.experimental.pallas import tpu as pltpu
from jax.experimental.pallas import tpu_sc as plsc

NC = 2
NS = 16
NW = NC * NS


def _gather_kernel(n_total, d, chunk):
    per_w = n_total // NW
    n_chunks = per_w // chunk
    mesh = plsc.VectorSubcoreMesh(core_axis_name="c", subcore_axis_name="s")

    @functools.partial(
        pl.kernel,
        mesh=mesh,
        out_type=jax.ShapeDtypeStruct((n_total, d), jnp.float32),
        scratch_types=[
            pltpu.VMEM((per_w,), jnp.int32),
            pltpu.VMEM((chunk, d), jnp.float32),
            pltpu.SemaphoreType.DMA,
        ],
    )
    def k(idx_hbm, table_hbm, out_hbm, idx_v, rows_v, gsem):
        wid = lax.axis_index("s") * NC + lax.axis_index("c")
        base = wid * per_w
        pltpu.sync_copy(idx_hbm.at[pl.ds(base, per_w)], idx_v)

        def body(i, carry):
            idx_sl = idx_v.at[pl.ds(i * chunk, chunk)]
            pltpu.async_copy(table_hbm.at[idx_sl], rows_v, gsem).wait()
            pltpu.sync_copy(rows_v, out_hbm.at[pl.ds(base + i * chunk, chunk)])
            return carry

        lax.fori_loop(0, n_chunks, body, 0)

    return k


def kernel(channel_ids, embedding_table):
    b, l = channel_ids.shape
    v, d = embedding_table.shape
    n_total = b * l
    idx_flat = channel_ids.reshape(n_total)
    out = _gather_kernel(n_total, d, 128)(idx_flat, embedding_table)
    return out.reshape(b, l, d)


# speedup vs baseline: 2.8434x; 1.0311x over previous
import functools

import jax
import jax.numpy as jnp
from jax import lax
from jax.experimental import pallas as pl
from jax.experimental.pallas import tpu as pltpu
from jax.experimental.pallas import tpu_sc as plsc

NC = 2
NS = 16
NW = NC * NS


def _gather_kernel(n_total, d, chunk):
    per_w = n_total // NW
    n_chunks = per_w // chunk
    assert n_chunks % 2 == 0 and n_chunks >= 6
    mesh = plsc.VectorSubcoreMesh(core_axis_name="c", subcore_axis_name="s")

    @functools.partial(
        pl.kernel,
        mesh=mesh,
        out_type=jax.ShapeDtypeStruct((n_total, d), jnp.float32),
        scratch_types=[
            pltpu.VMEM((per_w,), jnp.int32),
            pltpu.VMEM((2, chunk, d), jnp.float32),
            pltpu.SemaphoreType.DMA,
            pltpu.SemaphoreType.DMA,
            pltpu.SemaphoreType.DMA,
            pltpu.SemaphoreType.DMA,
        ],
    )
    def k(idx_hbm, table_hbm, out_hbm, idx_v, rows_v, g0, g1, s0, s1):
        wid = lax.axis_index("s") * NC + lax.axis_index("c")
        base = wid * per_w
        pltpu.sync_copy(idx_hbm.at[pl.ds(base, per_w)], idx_v)

        gsem = [g0, g1]
        ssem = [s0, s1]

        def g_desc(i, b):
            return pltpu.make_async_copy(
                table_hbm.at[idx_v.at[pl.ds(i * chunk, chunk)]],
                rows_v.at[b], gsem[b])

        def s_desc(i, b):
            return pltpu.make_async_copy(
                rows_v.at[b], out_hbm.at[pl.ds(base + i * chunk, chunk)],
                ssem[b])

        g_desc(0, 0).start()
        g_desc(0, 0).wait()
        g_desc(1, 1).start()
        s_desc(0, 0).start()
        g_desc(1, 1).wait()
        s_desc(0, 0).wait()
        g_desc(2, 0).start()
        s_desc(1, 1).start()

        def body(r, carry):
            for b in range(2):
                i = 2 + 2 * r + b
                g_desc(i, b).wait()
                s_desc(i - 1, 1 - b).wait()
                g_desc(i + 1, 1 - b).start()
                s_desc(i, b).start()
            return carry

        lax.fori_loop(0, (n_chunks - 4) // 2, body, 0)

        i = n_chunks - 2
        g_desc(i, 0).wait()
        s_desc(i - 1, 1).wait()
        g_desc(i + 1, 1).start()
        s_desc(i, 0).start()
        i = n_chunks - 1
        g_desc(i, 1).wait()
        s_desc(i - 1, 0).wait()
        s_desc(i, 1).start()
        s_desc(i, 1).wait()

    return k


def kernel(channel_ids, embedding_table):
    b, l = channel_ids.shape
    v, d = embedding_table.shape
    n_total = b * l
    idx_flat = channel_ids.reshape(n_total)
    out = _gather_kernel(n_total, d, 128)(idx_flat, embedding_table)
    return out.reshape(b, l, d)


# speedup vs baseline: 3.6523x; 1.2845x over previous
import functools

import jax
import jax.numpy as jnp
from jax import lax
from jax.experimental import pallas as pl
from jax.experimental.pallas import tpu as pltpu
from jax.experimental.pallas import tpu_sc as plsc

NC = 2
NS = 16
NW = NC * NS


def _gather_kernel(n_total, v_rows, d, chunk):
    per_w = n_total // NW
    n_chunks = per_w // chunk
    assert n_chunks % 2 == 0 and n_chunks >= 6
    mesh = plsc.VectorSubcoreMesh(core_axis_name="c", subcore_axis_name="s")

    @functools.partial(
        pl.kernel,
        mesh=mesh,
        out_type=jax.ShapeDtypeStruct((n_total, d), jnp.float32),
        scratch_types=[
            pltpu.VMEM((per_w,), jnp.int32),
            pltpu.VMEM((2, chunk, d), jnp.float32),
            pltpu.VMEM_SHARED((v_rows, d), jnp.float32),
            pltpu.SemaphoreType.DMA,
            pltpu.SemaphoreType.DMA,
            pltpu.SemaphoreType.DMA,
            pltpu.SemaphoreType.DMA,
        ],
    )
    def k(idx_hbm, table_hbm, out_hbm, idx_v, rows_v, spm_table, g0, g1, s0, s1):
        wid = lax.axis_index("s") * NC + lax.axis_index("c")
        base = wid * per_w
        @pl.when(lax.axis_index("s") == 0)
        def _stage():
            pltpu.sync_copy(table_hbm, spm_table)

        pltpu.sync_copy(idx_hbm.at[pl.ds(base, per_w)], idx_v)
        plsc.subcore_barrier()

        gsem = [g0, g1]
        ssem = [s0, s1]

        def g_desc(i, b):
            return pltpu.make_async_copy(
                spm_table.at[idx_v.at[pl.ds(i * chunk, chunk)]],
                rows_v.at[b], gsem[b])

        def s_desc(i, b):
            return pltpu.make_async_copy(
                rows_v.at[b], out_hbm.at[pl.ds(base + i * chunk, chunk)],
                ssem[b])

        g_desc(0, 0).start()
        g_desc(0, 0).wait()
        g_desc(1, 1).start()
        s_desc(0, 0).start()
        g_desc(1, 1).wait()
        s_desc(0, 0).wait()
        g_desc(2, 0).start()
        s_desc(1, 1).start()

        def body(r, carry):
            for b in range(2):
                i = 2 + 2 * r + b
                g_desc(i, b).wait()
                s_desc(i - 1, 1 - b).wait()
                g_desc(i + 1, 1 - b).start()
                s_desc(i, b).start()
            return carry

        lax.fori_loop(0, (n_chunks - 4) // 2, body, 0)

        i = n_chunks - 2
        g_desc(i, 0).wait()
        s_desc(i - 1, 1).wait()
        g_desc(i + 1, 1).start()
        s_desc(i, 0).start()
        i = n_chunks - 1
        g_desc(i, 1).wait()
        s_desc(i - 1, 0).wait()
        s_desc(i, 1).start()
        s_desc(i, 1).wait()

    return k


def kernel(channel_ids, embedding_table):
    b, l = channel_ids.shape
    v, d = embedding_table.shape
    n_total = b * l
    idx_flat = channel_ids.reshape(n_total)
    out = _gather_kernel(n_total, v, d, 128)(idx_flat, embedding_table)
    return out.reshape(b, l, d)
